# baseline (speedup 1.0000x reference)
import functools

import jax
import jax.numpy as jnp
from jax import lax
from jax.experimental import pallas as pl
from jax.experimental.pallas import tpu as pltpu
from jax.experimental.pallas import tpu_sc as plsc

_VOCAB = 100000
_EMBED = 128
_HIDDEN = 256
_MOE = 64
_NEXP = 8
_S = 2048
_L = 16

_NBLK = 512


@functools.lru_cache(maxsize=None)
def _sc_gather(V, D, B):
    info = plsc.get_sparse_core_info()
    nw = info.num_cores * info.num_subcores
    bpw = B // nw
    mesh = plsc.VectorSubcoreMesh(core_axis_name="c", subcore_axis_name="s")

    @functools.partial(
        pl.kernel,
        mesh=mesh,
        out_type=jax.ShapeDtypeStruct((B, D), jnp.float32),
        scratch_types=[
            pltpu.VMEM((bpw,), jnp.int32),
            pltpu.VMEM((bpw, D), jnp.float32),
            pltpu.SemaphoreType.DMA,
        ],
    )
    def g(table_hbm, idx_hbm, out_hbm, idx_v, rows_v, sem):
        wid = lax.axis_index("s") * info.num_cores + lax.axis_index("c")
        base = wid * bpw
        pltpu.sync_copy(idx_hbm.at[pl.ds(base, bpw)], idx_v)
        pltpu.async_copy(table_hbm.at[idx_v], rows_v, sem).wait()
        pltpu.sync_copy(rows_v, out_hbm.at[pl.ds(base, bpw)])

    return g


def _fdot(a, b):
    return lax.dot_general(a, b, (((1,), (0,)), ((), ())),
                           preferred_element_type=jnp.float32)


def _gif_stack(h):
    v = jnp.zeros_like(h)
    s_sum = jnp.zeros_like(h)
    for _ in range(_L):
        v = v + h
        s = jax.nn.sigmoid(5.0 * (v - 1.0))
        v = v - s
        s_sum = s_sum + s
    return s_sum * (1.0 / _L)


def _front_body(ids_ref, er_ref, encW_ref, encb_ref, s2cW_ref, s2cb_ref,
                rW1_ref, rb1_ref, rW2_ref, rb2_ref, eW1_ref, eb1_ref,
                eW2_ref, eb2_ref, c2sW_ref, c2sb_ref, decW_ref, decb_ref,
                dec_ref):
    ids = ids_ref[...]
    score = (ids % 97).astype(jnp.float32) * (1.0 / 97.0)
    it = lax.broadcasted_iota(jnp.int32, score.shape, 0)
    m1 = jnp.max(score, axis=0, keepdims=True)
    i1 = jnp.min(jnp.where(score == m1, it, _S), axis=0, keepdims=True)
    sel1 = it == i1
    m2 = jnp.max(jnp.where(sel1, -1.0, score), axis=0, keepdims=True)
    i2 = jnp.min(jnp.where((score == m2) & (~sel1), it, _S), axis=0,
                 keepdims=True)
    g = jnp.where(sel1 | (it == i2), 1.5, 1.0)

    mod = er_ref[...] * g
    spk = _gif_stack(_fdot(mod, encW_ref[...]) + encb_ref[...])
    cont = _fdot(spk, s2cW_ref[...]) + s2cb_ref[...]

    hr = jnp.tanh(_fdot(cont, rW1_ref[...]) + rb1_ref[...])
    rl = (_fdot(hr, rW2_ref[...]) + rb2_ref[...]) * g
    rmax = jnp.max(rl, axis=1, keepdims=True)
    e = jnp.exp(rl - rmax)
    p = e / jnp.sum(e, axis=1, keepdims=True)
    ie = lax.broadcasted_iota(jnp.int32, p.shape, 1)
    p1 = jnp.max(p, axis=1, keepdims=True)
    e1 = jnp.min(jnp.where(p == p1, ie, _NEXP), axis=1, keepdims=True)
    sele1 = ie == e1
    p2 = jnp.max(jnp.where(sele1, -1.0, p), axis=1, keepdims=True)
    e2 = jnp.min(jnp.where((p == p2) & (~sele1), ie, _NEXP), axis=1,
                 keepdims=True)
    denom = p1 + p2 + 1e-9
    w1 = p1 / denom
    w2 = p2 / denom

    acc = jnp.zeros_like(cont)
    for i in range(_NEXP):
        ex = _fdot(jax.nn.relu(_fdot(cont, eW1_ref[i]) + eb1_ref[i]),
                   eW2_ref[i]) + eb2_ref[i]
        wi = w1 * (e1 == i).astype(jnp.float32) + \
             w2 * (e2 == i).astype(jnp.float32)
        acc = acc + ex * wi

    rate = jax.nn.sigmoid(_fdot(acc, c2sW_ref[...]) + c2sb_ref[...])
    dec = _gif_stack(_fdot(rate * g, decW_ref[...]) + decb_ref[...])
    dec_ref[...] = dec


def _front(ids_col, emb_rows, enc_W, enc_b, s2c_W, s2c_b, rW1, rb1, rW2,
           rb2, eW1, eb1, eW2, eb2, c2s_W, c2s_b, dec_W, dec_b,
           interpret=False):
    return pl.pallas_call(
        _front_body,
        out_shape=jax.ShapeDtypeStruct((_S, _EMBED), jnp.float32),
        interpret=interpret,
    )(ids_col, emb_rows, enc_W, enc_b.reshape(1, -1), s2c_W,
      s2c_b.reshape(1, -1), rW1, rb1.reshape(1, -1), rW2,
      rb2.reshape(1, -1), eW1, eb1.reshape(_NEXP, 1, -1), eW2,
      eb2.reshape(_NEXP, 1, -1), c2s_W, c2s_b.reshape(1, -1), dec_W,
      dec_b.reshape(1, -1))


def _vocab_body(dec_ref, w_ref, b_ref, o_ref):
    o_ref[...] = lax.dot_general(
        dec_ref[...], w_ref[...], (((1,), (0,)), ((), ())),
        preferred_element_type=jnp.float32) + b_ref[...]


def _vocab(dec, out_W, out_b2d, interpret=False):
    grid = (pl.cdiv(_VOCAB, _NBLK),)
    return pl.pallas_call(
        _vocab_body,
        grid=grid,
        in_specs=[
            pl.BlockSpec((_S, _EMBED), lambda j: (0, 0)),
            pl.BlockSpec((_EMBED, _NBLK), lambda j: (0, j)),
            pl.BlockSpec((1, _NBLK), lambda j: (0, j)),
        ],
        out_specs=pl.BlockSpec((_S, _NBLK), lambda j: (0, j)),
        out_shape=jax.ShapeDtypeStruct((_S, _VOCAB), jnp.float32),
        compiler_params=pltpu.CompilerParams(
            dimension_semantics=("arbitrary",)),
        interpret=interpret,
    )(dec, out_W, out_b2d)


def kernel(input_ids, emb, enc_W, enc_b, s2c_W, s2c_b, rW1, rb1, rW2, rb2,
           eW1, eb1, eW2, eb2, c2s_W, c2s_b, dec_W, dec_b, out_W, out_b):
    bsz, seq = input_ids.shape
    ids_flat = input_ids.reshape(-1)
    emb_rows = _sc_gather(_VOCAB, _EMBED, bsz * seq)(emb, ids_flat)
    dec = _front(ids_flat.reshape(-1, 1), emb_rows, enc_W, enc_b, s2c_W,
                 s2c_b, rW1, rb1, rW2, rb2, eW1, eb1, eW2, eb2, c2s_W,
                 c2s_b, dec_W, dec_b)
    logits = _vocab(dec, out_W, out_b.reshape(1, -1))
    return logits.reshape(bsz, seq, _VOCAB)


# speedup vs baseline: 2.0518x; 2.0518x over previous
import functools

import jax
import jax.numpy as jnp
from jax import lax
from jax.experimental import pallas as pl
from jax.experimental.pallas import tpu as pltpu
from jax.experimental.pallas import tpu_sc as plsc

_VOCAB = 100000
_EMBED = 128
_HIDDEN = 256
_MOE = 64
_NEXP = 8
_S = 2048
_L = 16

_NBLK = 512


@functools.lru_cache(maxsize=None)
def _sc_gather(V, D, B):
    info = plsc.get_sparse_core_info()
    nw = info.num_cores * info.num_subcores
    bpw = B // nw
    mesh = plsc.VectorSubcoreMesh(core_axis_name="c", subcore_axis_name="s")

    @functools.partial(
        pl.kernel,
        mesh=mesh,
        out_type=jax.ShapeDtypeStruct((B, D), jnp.float32),
        scratch_types=[
            pltpu.VMEM((bpw,), jnp.int32),
            pltpu.VMEM((bpw, D), jnp.float32),
            pltpu.SemaphoreType.DMA,
        ],
        compiler_params=pltpu.CompilerParams(use_tc_tiling_on_sc=True),
    )
    def g(table_hbm, idx_hbm, out_hbm, idx_v, rows_v, sem):
        wid = lax.axis_index("s") * info.num_cores + lax.axis_index("c")
        base = wid * bpw
        pltpu.sync_copy(idx_hbm.at[pl.ds(base, bpw)], idx_v)
        pltpu.async_copy(table_hbm.at[idx_v], rows_v, sem).wait()
        pltpu.sync_copy(rows_v, out_hbm.at[pl.ds(base, bpw)])

    return g


def _fdot(a, b):
    return lax.dot_general(a, b, (((1,), (0,)), ((), ())),
                           preferred_element_type=jnp.float32)


def _gif_stack(h):
    v = jnp.zeros_like(h)
    s_sum = jnp.zeros_like(h)
    for _ in range(_L):
        v = v + h
        s = jax.nn.sigmoid(5.0 * (v - 1.0))
        v = v - s
        s_sum = s_sum + s
    return s_sum * (1.0 / _L)


def _front_body(ids_ref, er_ref, encW_ref, encb_ref, s2cW_ref, s2cb_ref,
                rW1_ref, rb1_ref, rW2_ref, rb2_ref, eW1_ref, eb1_ref,
                eW2_ref, eb2_ref, c2sW_ref, c2sb_ref, decW_ref, decb_ref,
                dec_ref):
    ids = ids_ref[...]
    score = (ids % 97).astype(jnp.float32) * (1.0 / 97.0)
    it = lax.broadcasted_iota(jnp.int32, score.shape, 0)
    m1 = jnp.max(score, axis=0, keepdims=True)
    i1 = jnp.min(jnp.where(score == m1, it, _S), axis=0, keepdims=True)
    sel1 = it == i1
    m2 = jnp.max(jnp.where(sel1, -1.0, score), axis=0, keepdims=True)
    i2 = jnp.min(jnp.where((score == m2) & (~sel1), it, _S), axis=0,
                 keepdims=True)
    g = jnp.where(sel1 | (it == i2), 1.5, 1.0)

    mod = er_ref[...] * g
    spk = _gif_stack(_fdot(mod, encW_ref[...]) + encb_ref[...])
    cont = _fdot(spk, s2cW_ref[...]) + s2cb_ref[...]

    hr = jnp.tanh(_fdot(cont, rW1_ref[...]) + rb1_ref[...])
    rl = (_fdot(hr, rW2_ref[...]) + rb2_ref[...]) * g
    rmax = jnp.max(rl, axis=1, keepdims=True)
    e = jnp.exp(rl - rmax)
    p = e / jnp.sum(e, axis=1, keepdims=True)
    ie = lax.broadcasted_iota(jnp.int32, p.shape, 1)
    p1 = jnp.max(p, axis=1, keepdims=True)
    e1 = jnp.min(jnp.where(p == p1, ie, _NEXP), axis=1, keepdims=True)
    sele1 = ie == e1
    p2 = jnp.max(jnp.where(sele1, -1.0, p), axis=1, keepdims=True)
    e2 = jnp.min(jnp.where((p == p2) & (~sele1), ie, _NEXP), axis=1,
                 keepdims=True)
    denom = p1 + p2 + 1e-9
    w1 = p1 / denom
    w2 = p2 / denom

    acc = jnp.zeros_like(cont)
    for i in range(_NEXP):
        ex = _fdot(jax.nn.relu(_fdot(cont, eW1_ref[i]) + eb1_ref[i]),
                   eW2_ref[i]) + eb2_ref[i]
        wi = w1 * (e1 == i).astype(jnp.float32) + \
             w2 * (e2 == i).astype(jnp.float32)
        acc = acc + ex * wi

    rate = jax.nn.sigmoid(_fdot(acc, c2sW_ref[...]) + c2sb_ref[...])
    dec = _gif_stack(_fdot(rate * g, decW_ref[...]) + decb_ref[...])
    dec_ref[...] = dec


def _front(ids_col, emb_rows, enc_W, enc_b, s2c_W, s2c_b, rW1, rb1, rW2,
           rb2, eW1, eb1, eW2, eb2, c2s_W, c2s_b, dec_W, dec_b,
           interpret=False):
    return pl.pallas_call(
        _front_body,
        out_shape=jax.ShapeDtypeStruct((_S, _EMBED), jnp.float32),
        interpret=interpret,
    )(ids_col, emb_rows, enc_W, enc_b.reshape(1, -1), s2c_W,
      s2c_b.reshape(1, -1), rW1, rb1.reshape(1, -1), rW2,
      rb2.reshape(1, -1), eW1, eb1.reshape(_NEXP, 1, -1), eW2,
      eb2.reshape(_NEXP, 1, -1), c2s_W, c2s_b.reshape(1, -1), dec_W,
      dec_b.reshape(1, -1))


def _vocab_body(dec_ref, w_ref, b_ref, o_ref):
    o_ref[...] = lax.dot_general(
        w_ref[...], dec_ref[...], (((0,), (1,)), ((), ())),
        preferred_element_type=jnp.float32) + b_ref[...]


def _vocab(dec, out_W, out_b2d, interpret=False):
    grid = (pl.cdiv(_VOCAB, _NBLK),)
    return pl.pallas_call(
        _vocab_body,
        grid=grid,
        in_specs=[
            pl.BlockSpec((_S, _EMBED), lambda j: (0, 0)),
            pl.BlockSpec((_EMBED, _NBLK), lambda j: (0, j)),
            pl.BlockSpec((_NBLK, 1), lambda j: (j, 0)),
        ],
        out_specs=pl.BlockSpec((_NBLK, _S), lambda j: (j, 0)),
        out_shape=jax.ShapeDtypeStruct((_VOCAB, _S), jnp.float32),
        compiler_params=pltpu.CompilerParams(
            dimension_semantics=("arbitrary",)),
        interpret=interpret,
    )(dec, out_W, out_b2d)


def kernel(input_ids, emb, enc_W, enc_b, s2c_W, s2c_b, rW1, rb1, rW2, rb2,
           eW1, eb1, eW2, eb2, c2s_W, c2s_b, dec_W, dec_b, out_W, out_b):
    bsz, seq = input_ids.shape
    ids_flat = input_ids.reshape(-1)
    emb_rows = _sc_gather(_VOCAB, _EMBED, bsz * seq)(emb, ids_flat)
    dec = _front(ids_flat.reshape(-1, 1), emb_rows, enc_W, enc_b, s2c_W,
                 s2c_b, rW1, rb1, rW2, rb2, eW1, eb1, eW2, eb2, c2s_W,
                 c2s_b, dec_W, dec_b)
    logits_t = _vocab(dec, out_W, out_b.reshape(-1, 1))
    return logits_t.T.reshape(bsz, seq, _VOCAB)


# speedup vs baseline: 2.2720x; 1.1073x over previous
import functools

import jax
import jax.numpy as jnp
from jax import lax
from jax.experimental import pallas as pl
from jax.experimental.pallas import tpu as pltpu
from jax.experimental.pallas import tpu_sc as plsc

_VOCAB = 100000
_EMBED = 128
_HIDDEN = 256
_MOE = 64
_NEXP = 8
_S = 2048
_L = 16

_NBLK = 1024


@functools.lru_cache(maxsize=None)
def _sc_gather(V, D, B):
    info = plsc.get_sparse_core_info()
    nw = info.num_cores * info.num_subcores
    bpw = B // nw
    mesh = plsc.VectorSubcoreMesh(core_axis_name="c", subcore_axis_name="s")

    @functools.partial(
        pl.kernel,
        mesh=mesh,
        out_type=jax.ShapeDtypeStruct((B, D), jnp.float32),
        scratch_types=[
            pltpu.VMEM((bpw,), jnp.int32),
            pltpu.VMEM((bpw, D), jnp.float32),
            pltpu.SemaphoreType.DMA,
        ],
        compiler_params=pltpu.CompilerParams(use_tc_tiling_on_sc=True),
    )
    def g(table_hbm, idx_hbm, out_hbm, idx_v, rows_v, sem):
        wid = lax.axis_index("s") * info.num_cores + lax.axis_index("c")
        base = wid * bpw
        pltpu.sync_copy(idx_hbm.at[pl.ds(base, bpw)], idx_v)
        pltpu.async_copy(table_hbm.at[idx_v], rows_v, sem).wait()
        pltpu.sync_copy(rows_v, out_hbm.at[pl.ds(base, bpw)])

    return g


def _fdot(a, b):
    return lax.dot_general(a, b, (((1,), (0,)), ((), ())),
                           preferred_element_type=jnp.float32)


def _gif_stack(h):
    v = jnp.zeros_like(h)
    s_sum = jnp.zeros_like(h)
    for _ in range(_L):
        v = v + h
        s = jax.nn.sigmoid(5.0 * (v - 1.0))
        v = v - s
        s_sum = s_sum + s
    return s_sum * (1.0 / _L)


def _front_body(ids_ref, er_ref, encW_ref, encb_ref, s2cW_ref, s2cb_ref,
                rW1_ref, rb1_ref, rW2_ref, rb2_ref, eW1_ref, eb1_ref,
                eW2_ref, eb2_ref, c2sW_ref, c2sb_ref, decW_ref, decb_ref,
                dec_ref):
    ids = ids_ref[...]
    score = (ids % 97).astype(jnp.float32) * (1.0 / 97.0)
    it = lax.broadcasted_iota(jnp.int32, score.shape, 0)
    m1 = jnp.max(score, axis=0, keepdims=True)
    i1 = jnp.min(jnp.where(score == m1, it, _S), axis=0, keepdims=True)
    sel1 = it == i1
    m2 = jnp.max(jnp.where(sel1, -1.0, score), axis=0, keepdims=True)
    i2 = jnp.min(jnp.where((score == m2) & (~sel1), it, _S), axis=0,
                 keepdims=True)
    g = jnp.where(sel1 | (it == i2), 1.5, 1.0)

    mod = er_ref[...] * g
    spk = _gif_stack(_fdot(mod, encW_ref[...]) + encb_ref[...])
    cont = _fdot(spk, s2cW_ref[...]) + s2cb_ref[...]

    hr = jnp.tanh(_fdot(cont, rW1_ref[...]) + rb1_ref[...])
    rl = (_fdot(hr, rW2_ref[...]) + rb2_ref[...]) * g
    rmax = jnp.max(rl, axis=1, keepdims=True)
    e = jnp.exp(rl - rmax)
    p = e / jnp.sum(e, axis=1, keepdims=True)
    ie = lax.broadcasted_iota(jnp.int32, p.shape, 1)
    p1 = jnp.max(p, axis=1, keepdims=True)
    e1 = jnp.min(jnp.where(p == p1, ie, _NEXP), axis=1, keepdims=True)
    sele1 = ie == e1
    p2 = jnp.max(jnp.where(sele1, -1.0, p), axis=1, keepdims=True)
    e2 = jnp.min(jnp.where((p == p2) & (~sele1), ie, _NEXP), axis=1,
                 keepdims=True)
    denom = p1 + p2 + 1e-9
    w1 = p1 / denom
    w2 = p2 / denom

    acc = jnp.zeros_like(cont)
    for i in range(_NEXP):
        ex = _fdot(jax.nn.relu(_fdot(cont, eW1_ref[i]) + eb1_ref[i]),
                   eW2_ref[i]) + eb2_ref[i]
        wi = w1 * (e1 == i).astype(jnp.float32) + \
             w2 * (e2 == i).astype(jnp.float32)
        acc = acc + ex * wi

    rate = jax.nn.sigmoid(_fdot(acc, c2sW_ref[...]) + c2sb_ref[...])
    dec = _gif_stack(_fdot(rate * g, decW_ref[...]) + decb_ref[...])
    dec_ref[...] = dec


def _front(ids_col, emb_rows, enc_W, enc_b, s2c_W, s2c_b, rW1, rb1, rW2,
           rb2, eW1, eb1, eW2, eb2, c2s_W, c2s_b, dec_W, dec_b,
           interpret=False):
    return pl.pallas_call(
        _front_body,
        out_shape=jax.ShapeDtypeStruct((_S, _EMBED), jnp.float32),
        interpret=interpret,
    )(ids_col, emb_rows, enc_W, enc_b.reshape(1, -1), s2c_W,
      s2c_b.reshape(1, -1), rW1, rb1.reshape(1, -1), rW2,
      rb2.reshape(1, -1), eW1, eb1.reshape(_NEXP, 1, -1), eW2,
      eb2.reshape(_NEXP, 1, -1), c2s_W, c2s_b.reshape(1, -1), dec_W,
      dec_b.reshape(1, -1))


def _vocab_body(dec_ref, w_ref, b_ref, o_ref):
    o_ref[...] = lax.dot_general(
        w_ref[...], dec_ref[...], (((0,), (1,)), ((), ())),
        preferred_element_type=jnp.float32) + b_ref[...]


def _vocab(dec, out_W, out_b2d, interpret=False):
    grid = (pl.cdiv(_VOCAB, _NBLK),)
    return pl.pallas_call(
        _vocab_body,
        grid=grid,
        in_specs=[
            pl.BlockSpec((_S, _EMBED), lambda j: (0, 0)),
            pl.BlockSpec((_EMBED, _NBLK), lambda j: (0, j)),
            pl.BlockSpec((_NBLK, 1), lambda j: (j, 0)),
        ],
        out_specs=pl.BlockSpec((_NBLK, _S), lambda j: (j, 0)),
        out_shape=jax.ShapeDtypeStruct((_VOCAB, _S), jnp.float32),
        compiler_params=pltpu.CompilerParams(
            dimension_semantics=("arbitrary",)),
        interpret=interpret,
    )(dec, out_W, out_b2d)


def kernel(input_ids, emb, enc_W, enc_b, s2c_W, s2c_b, rW1, rb1, rW2, rb2,
           eW1, eb1, eW2, eb2, c2s_W, c2s_b, dec_W, dec_b, out_W, out_b):
    bsz, seq = input_ids.shape
    ids_flat = input_ids.reshape(-1)
    emb_rows = _sc_gather(_VOCAB, _EMBED, bsz * seq)(emb, ids_flat)
    dec = _front(ids_flat.reshape(-1, 1), emb_rows, enc_W, enc_b, s2c_W,
                 s2c_b, rW1, rb1, rW2, rb2, eW1, eb1, eW2, eb2, c2s_W,
                 c2s_b, dec_W, dec_b)
    logits_t = _vocab(dec, out_W, out_b.reshape(-1, 1))
    return logits_t.T.reshape(bsz, seq, _VOCAB)


# speedup vs baseline: 2.3237x; 1.0227x over previous
import functools

import jax
import jax.numpy as jnp
from jax import lax
from jax.experimental import pallas as pl
from jax.experimental.pallas import tpu as pltpu
from jax.experimental.pallas import tpu_sc as plsc

_VOCAB = 100000
_EMBED = 128
_HIDDEN = 256
_MOE = 64
_NEXP = 8
_S = 2048
_L = 16

_NBLK = 2048


@functools.lru_cache(maxsize=None)
def _sc_gather(V, D, B):
    info = plsc.get_sparse_core_info()
    nw = info.num_cores * info.num_subcores
    bpw = B // nw
    mesh = plsc.VectorSubcoreMesh(core_axis_name="c", subcore_axis_name="s")

    @functools.partial(
        pl.kernel,
        mesh=mesh,
        out_type=jax.ShapeDtypeStruct((B, D), jnp.float32),
        scratch_types=[
            pltpu.VMEM((bpw,), jnp.int32),
            pltpu.VMEM((bpw, D), jnp.float32),
            pltpu.SemaphoreType.DMA,
        ],
        compiler_params=pltpu.CompilerParams(use_tc_tiling_on_sc=True),
    )
    def g(table_hbm, idx_hbm, out_hbm, idx_v, rows_v, sem):
        wid = lax.axis_index("s") * info.num_cores + lax.axis_index("c")
        base = wid * bpw
        pltpu.sync_copy(idx_hbm.at[pl.ds(base, bpw)], idx_v)
        pltpu.async_copy(table_hbm.at[idx_v], rows_v, sem).wait()
        pltpu.sync_copy(rows_v, out_hbm.at[pl.ds(base, bpw)])

    return g


def _fdot(a, b):
    return lax.dot_general(a, b, (((1,), (0,)), ((), ())),
                           preferred_element_type=jnp.float32)


def _gif_stack(h):
    v = jnp.zeros_like(h)
    s_sum = jnp.zeros_like(h)
    for _ in range(_L):
        v = v + h
        s = jax.nn.sigmoid(5.0 * (v - 1.0))
        v = v - s
        s_sum = s_sum + s
    return s_sum * (1.0 / _L)


def _front_body(ids_ref, er_ref, encW_ref, encb_ref, s2cW_ref, s2cb_ref,
                rW1_ref, rb1_ref, rW2_ref, rb2_ref, eW1_ref, eb1_ref,
                eW2_ref, eb2_ref, c2sW_ref, c2sb_ref, decW_ref, decb_ref,
                dec_ref):
    ids = ids_ref[...]
    score = (ids % 97).astype(jnp.float32) * (1.0 / 97.0)
    it = lax.broadcasted_iota(jnp.int32, score.shape, 0)
    m1 = jnp.max(score, axis=0, keepdims=True)
    i1 = jnp.min(jnp.where(score == m1, it, _S), axis=0, keepdims=True)
    sel1 = it == i1
    m2 = jnp.max(jnp.where(sel1, -1.0, score), axis=0, keepdims=True)
    i2 = jnp.min(jnp.where((score == m2) & (~sel1), it, _S), axis=0,
                 keepdims=True)
    g = jnp.where(sel1 | (it == i2), 1.5, 1.0)

    mod = er_ref[...] * g
    spk = _gif_stack(_fdot(mod, encW_ref[...]) + encb_ref[...])
    cont = _fdot(spk, s2cW_ref[...]) + s2cb_ref[...]

    hr = jnp.tanh(_fdot(cont, rW1_ref[...]) + rb1_ref[...])
    rl = (_fdot(hr, rW2_ref[...]) + rb2_ref[...]) * g
    rmax = jnp.max(rl, axis=1, keepdims=True)
    e = jnp.exp(rl - rmax)
    p = e / jnp.sum(e, axis=1, keepdims=True)
    ie = lax.broadcasted_iota(jnp.int32, p.shape, 1)
    p1 = jnp.max(p, axis=1, keepdims=True)
    e1 = jnp.min(jnp.where(p == p1, ie, _NEXP), axis=1, keepdims=True)
    sele1 = ie == e1
    p2 = jnp.max(jnp.where(sele1, -1.0, p), axis=1, keepdims=True)
    e2 = jnp.min(jnp.where((p == p2) & (~sele1), ie, _NEXP), axis=1,
                 keepdims=True)
    denom = p1 + p2 + 1e-9
    w1 = p1 / denom
    w2 = p2 / denom

    acc = jnp.zeros_like(cont)
    for i in range(_NEXP):
        ex = _fdot(jax.nn.relu(_fdot(cont, eW1_ref[i]) + eb1_ref[i]),
                   eW2_ref[i]) + eb2_ref[i]
        wi = w1 * (e1 == i).astype(jnp.float32) + \
             w2 * (e2 == i).astype(jnp.float32)
        acc = acc + ex * wi

    rate = jax.nn.sigmoid(_fdot(acc, c2sW_ref[...]) + c2sb_ref[...])
    dec = _gif_stack(_fdot(rate * g, decW_ref[...]) + decb_ref[...])
    dec_ref[...] = dec


def _front(ids_col, emb_rows, enc_W, enc_b, s2c_W, s2c_b, rW1, rb1, rW2,
           rb2, eW1, eb1, eW2, eb2, c2s_W, c2s_b, dec_W, dec_b,
           interpret=False):
    return pl.pallas_call(
        _front_body,
        out_shape=jax.ShapeDtypeStruct((_S, _EMBED), jnp.float32),
        interpret=interpret,
    )(ids_col, emb_rows, enc_W, enc_b.reshape(1, -1), s2c_W,
      s2c_b.reshape(1, -1), rW1, rb1.reshape(1, -1), rW2,
      rb2.reshape(1, -1), eW1, eb1.reshape(_NEXP, 1, -1), eW2,
      eb2.reshape(_NEXP, 1, -1), c2s_W, c2s_b.reshape(1, -1), dec_W,
      dec_b.reshape(1, -1))


def _vocab_body(dec_ref, w_ref, b_ref, o_ref):
    o_ref[...] = lax.dot_general(
        w_ref[...], dec_ref[...], (((0,), (1,)), ((), ())),
        preferred_element_type=jnp.float32) + b_ref[...]


def _vocab(dec, out_W, out_b2d, interpret=False):
    grid = (pl.cdiv(_VOCAB, _NBLK),)
    return pl.pallas_call(
        _vocab_body,
        grid=grid,
        in_specs=[
            pl.BlockSpec((_S, _EMBED), lambda j: (0, 0)),
            pl.BlockSpec((_EMBED, _NBLK), lambda j: (0, j)),
            pl.BlockSpec((_NBLK, 1), lambda j: (j, 0)),
        ],
        out_specs=pl.BlockSpec((_NBLK, _S), lambda j: (j, 0)),
        out_shape=jax.ShapeDtypeStruct((_VOCAB, _S), jnp.float32),
        compiler_params=pltpu.CompilerParams(
            dimension_semantics=("arbitrary",)),
        interpret=interpret,
    )(dec, out_W, out_b2d)


def kernel(input_ids, emb, enc_W, enc_b, s2c_W, s2c_b, rW1, rb1, rW2, rb2,
           eW1, eb1, eW2, eb2, c2s_W, c2s_b, dec_W, dec_b, out_W, out_b):
    bsz, seq = input_ids.shape
    ids_flat = input_ids.reshape(-1)
    emb_rows = _sc_gather(_VOCAB, _EMBED, bsz * seq)(emb, ids_flat)
    dec = _front(ids_flat.reshape(-1, 1), emb_rows, enc_W, enc_b, s2c_W,
                 s2c_b, rW1, rb1, rW2, rb2, eW1, eb1, eW2, eb2, c2s_W,
                 c2s_b, dec_W, dec_b)
    logits_t = _vocab(dec, out_W, out_b.reshape(-1, 1))
    return logits_t.T.reshape(bsz, seq, _VOCAB)


# speedup vs baseline: 3.0939x; 1.3315x over previous
import functools

import jax
import jax.numpy as jnp
from jax import lax
from jax.experimental import pallas as pl
from jax.experimental.pallas import tpu as pltpu
from jax.experimental.pallas import tpu_sc as plsc

_VOCAB = 100000
_EMBED = 128
_HIDDEN = 256
_MOE = 64
_NEXP = 8
_S = 2048
_L = 16

_NBLK = 2048


@functools.lru_cache(maxsize=None)
def _sc_gather(V, D, B):
    info = plsc.get_sparse_core_info()
    nw = info.num_cores * info.num_subcores
    bpw = B // nw
    mesh = plsc.VectorSubcoreMesh(core_axis_name="c", subcore_axis_name="s")

    @functools.partial(
        pl.kernel,
        mesh=mesh,
        out_type=jax.ShapeDtypeStruct((B, D), jnp.float32),
        scratch_types=[
            pltpu.VMEM((bpw,), jnp.int32),
            pltpu.VMEM((bpw, D), jnp.float32),
            pltpu.SemaphoreType.DMA,
        ],
        compiler_params=pltpu.CompilerParams(use_tc_tiling_on_sc=True),
    )
    def g(table_hbm, idx_hbm, out_hbm, idx_v, rows_v, sem):
        wid = lax.axis_index("s") * info.num_cores + lax.axis_index("c")
        base = wid * bpw
        pltpu.sync_copy(idx_hbm.at[pl.ds(base, bpw)], idx_v)
        pltpu.async_copy(table_hbm.at[idx_v], rows_v, sem).wait()
        pltpu.sync_copy(rows_v, out_hbm.at[pl.ds(base, bpw)])

    return g


def _fdot(a, b):
    return lax.dot_general(a, b, (((1,), (0,)), ((), ())),
                           preferred_element_type=jnp.float32)


def _gif_stack(h):
    v = jnp.zeros_like(h)
    s_sum = jnp.zeros_like(h)
    for _ in range(_L):
        v = v + h
        s = jax.nn.sigmoid(5.0 * (v - 1.0))
        v = v - s
        s_sum = s_sum + s
    return s_sum * (1.0 / _L)


def _front_body(ids_ref, er_ref, encW_ref, encb_ref, s2cW_ref, s2cb_ref,
                rW1_ref, rb1_ref, rW2_ref, rb2_ref, eW1_ref, eb1_ref,
                eW2_ref, eb2_ref, c2sW_ref, c2sb_ref, decW_ref, decb_ref,
                dec_ref):
    ids = ids_ref[...]
    score = (ids % 97).astype(jnp.float32) * (1.0 / 97.0)
    it = lax.broadcasted_iota(jnp.int32, score.shape, 0)
    m1 = jnp.max(score, axis=0, keepdims=True)
    i1 = jnp.min(jnp.where(score == m1, it, _S), axis=0, keepdims=True)
    sel1 = it == i1
    m2 = jnp.max(jnp.where(sel1, -1.0, score), axis=0, keepdims=True)
    i2 = jnp.min(jnp.where((score == m2) & (~sel1), it, _S), axis=0,
                 keepdims=True)
    g = jnp.where(sel1 | (it == i2), 1.5, 1.0)

    mod = er_ref[...] * g
    spk = _gif_stack(_fdot(mod, encW_ref[...]) + encb_ref[...])
    cont = _fdot(spk, s2cW_ref[...]) + s2cb_ref[...]

    hr = jnp.tanh(_fdot(cont, rW1_ref[...]) + rb1_ref[...])
    rl = (_fdot(hr, rW2_ref[...]) + rb2_ref[...]) * g
    rmax = jnp.max(rl, axis=1, keepdims=True)
    e = jnp.exp(rl - rmax)
    p = e / jnp.sum(e, axis=1, keepdims=True)
    ie = lax.broadcasted_iota(jnp.int32, p.shape, 1)
    p1 = jnp.max(p, axis=1, keepdims=True)
    e1 = jnp.min(jnp.where(p == p1, ie, _NEXP), axis=1, keepdims=True)
    sele1 = ie == e1
    p2 = jnp.max(jnp.where(sele1, -1.0, p), axis=1, keepdims=True)
    e2 = jnp.min(jnp.where((p == p2) & (~sele1), ie, _NEXP), axis=1,
                 keepdims=True)
    denom = p1 + p2 + 1e-9
    w1 = p1 / denom
    w2 = p2 / denom

    acc = jnp.zeros_like(cont)
    for i in range(_NEXP):
        ex = _fdot(jax.nn.relu(_fdot(cont, eW1_ref[i]) + eb1_ref[i]),
                   eW2_ref[i]) + eb2_ref[i]
        wi = w1 * (e1 == i).astype(jnp.float32) + \
             w2 * (e2 == i).astype(jnp.float32)
        acc = acc + ex * wi

    rate = jax.nn.sigmoid(_fdot(acc, c2sW_ref[...]) + c2sb_ref[...])
    dec = _gif_stack(_fdot(rate * g, decW_ref[...]) + decb_ref[...])
    dec_ref[...] = dec


def _front(ids_col, emb_rows, enc_W, enc_b, s2c_W, s2c_b, rW1, rb1, rW2,
           rb2, eW1, eb1, eW2, eb2, c2s_W, c2s_b, dec_W, dec_b,
           interpret=False):
    return pl.pallas_call(
        _front_body,
        out_shape=jax.ShapeDtypeStruct((_S, _EMBED), jnp.float32),
        interpret=interpret,
    )(ids_col, emb_rows, enc_W, enc_b.reshape(1, -1), s2c_W,
      s2c_b.reshape(1, -1), rW1, rb1.reshape(1, -1), rW2,
      rb2.reshape(1, -1), eW1, eb1.reshape(_NEXP, 1, -1), eW2,
      eb2.reshape(_NEXP, 1, -1), c2s_W, c2s_b.reshape(1, -1), dec_W,
      dec_b.reshape(1, -1))


def _vocab_body(dec_ref, wt_ref, b_ref, o_ref):
    b_col = jnp.transpose(b_ref[...], (1, 0))
    o_ref[...] = lax.dot_general(
        wt_ref[...], dec_ref[...], (((1,), (1,)), ((), ())),
        preferred_element_type=jnp.float32) + b_col


def _vocab(dec, out_Wt, out_b2d, interpret=False):
    grid = (pl.cdiv(_VOCAB, _NBLK),)
    return pl.pallas_call(
        _vocab_body,
        grid=grid,
        in_specs=[
            pl.BlockSpec((_S, _EMBED), lambda j: (0, 0)),
            pl.BlockSpec((_NBLK, _EMBED), lambda j: (j, 0)),
            pl.BlockSpec((1, _NBLK), lambda j: (0, j)),
        ],
        out_specs=pl.BlockSpec((_NBLK, _S), lambda j: (j, 0)),
        out_shape=jax.ShapeDtypeStruct((_VOCAB, _S), jnp.float32),
        compiler_params=pltpu.CompilerParams(
            dimension_semantics=("arbitrary",),
            vmem_limit_bytes=56 * 1024 * 1024),
        interpret=interpret,
    )(dec, out_Wt, out_b2d)


def kernel(input_ids, emb, enc_W, enc_b, s2c_W, s2c_b, rW1, rb1, rW2, rb2,
           eW1, eb1, eW2, eb2, c2s_W, c2s_b, dec_W, dec_b, out_W, out_b):
    bsz, seq = input_ids.shape
    ids_flat = input_ids.reshape(-1)
    emb_rows = _sc_gather(_VOCAB, _EMBED, bsz * seq)(emb, ids_flat)
    dec = _front(ids_flat.reshape(-1, 1), emb_rows, enc_W, enc_b, s2c_W,
                 s2c_b, rW1, rb1, rW2, rb2, eW1, eb1, eW2, eb2, c2s_W,
                 c2s_b, dec_W, dec_b)
    logits_t = _vocab(dec, out_W.T, out_b.reshape(1, -1))
    return logits_t.T.reshape(bsz, seq, _VOCAB)


# speedup vs baseline: 3.1558x; 1.0200x over previous
import functools

import jax
import jax.numpy as jnp
from jax import lax
from jax.experimental import pallas as pl
from jax.experimental.pallas import tpu as pltpu
from jax.experimental.pallas import tpu_sc as plsc

_VOCAB = 100000
_EMBED = 128
_HIDDEN = 256
_MOE = 64
_NEXP = 8
_S = 2048
_L = 16

_NBLK = 2048


@functools.lru_cache(maxsize=None)
def _sc_gather(V, D, B):
    info = plsc.get_sparse_core_info()
    nw = info.num_cores * info.num_subcores
    bpw = B // nw
    mesh = plsc.VectorSubcoreMesh(core_axis_name="c", subcore_axis_name="s")

    @functools.partial(
        pl.kernel,
        mesh=mesh,
        out_type=jax.ShapeDtypeStruct((B, D), jnp.float32),
        scratch_types=[
            pltpu.VMEM((bpw,), jnp.int32),
            pltpu.VMEM((bpw, D), jnp.float32),
            pltpu.SemaphoreType.DMA,
        ],
        compiler_params=pltpu.CompilerParams(use_tc_tiling_on_sc=True),
    )
    def g(table_hbm, idx_hbm, out_hbm, idx_v, rows_v, sem):
        wid = lax.axis_index("s") * info.num_cores + lax.axis_index("c")
        base = wid * bpw
        pltpu.sync_copy(idx_hbm.at[pl.ds(base, bpw)], idx_v)
        pltpu.async_copy(table_hbm.at[idx_v], rows_v, sem).wait()
        pltpu.sync_copy(rows_v, out_hbm.at[pl.ds(base, bpw)])

    return g


def _fdot(a, b):
    return lax.dot_general(a, b, (((1,), (0,)), ((), ())),
                           preferred_element_type=jnp.float32)


def _gif_stack(h):
    h25 = 2.5 * h
    w = jnp.full_like(h, -2.5)
    th_sum = jnp.zeros_like(h)
    for _ in range(_L):
        w = w + h25
        th = jnp.tanh(w)
        th_sum = th_sum + th
        w = w - (1.25 * th + 1.25)
    return th_sum * (1.0 / (2 * _L)) + 0.5


def _front_body(ids_ref, er_ref, encW_ref, encb_ref, s2cW_ref, s2cb_ref,
                rW1_ref, rb1_ref, rW2_ref, rb2_ref, eW1_ref, eb1_ref,
                eW2_ref, eb2_ref, c2sW_ref, c2sb_ref, decW_ref, decb_ref,
                dec_ref):
    ids = ids_ref[...]
    score = (ids % 97).astype(jnp.float32) * (1.0 / 97.0)
    it = lax.broadcasted_iota(jnp.int32, score.shape, 0)
    m1 = jnp.max(score, axis=0, keepdims=True)
    i1 = jnp.min(jnp.where(score == m1, it, _S), axis=0, keepdims=True)
    sel1 = it == i1
    m2 = jnp.max(jnp.where(sel1, -1.0, score), axis=0, keepdims=True)
    i2 = jnp.min(jnp.where((score == m2) & (~sel1), it, _S), axis=0,
                 keepdims=True)
    g = jnp.where(sel1 | (it == i2), 1.5, 1.0)

    mod = er_ref[...] * g
    spk = _gif_stack(_fdot(mod, encW_ref[...]) + encb_ref[...])
    cont = _fdot(spk, s2cW_ref[...]) + s2cb_ref[...]

    hr = jnp.tanh(_fdot(cont, rW1_ref[...]) + rb1_ref[...])
    rl = (_fdot(hr, rW2_ref[...]) + rb2_ref[...]) * g
    rmax = jnp.max(rl, axis=1, keepdims=True)
    e = jnp.exp(rl - rmax)
    p = e / jnp.sum(e, axis=1, keepdims=True)
    ie = lax.broadcasted_iota(jnp.int32, p.shape, 1)
    p1 = jnp.max(p, axis=1, keepdims=True)
    e1 = jnp.min(jnp.where(p == p1, ie, _NEXP), axis=1, keepdims=True)
    sele1 = ie == e1
    p2 = jnp.max(jnp.where(sele1, -1.0, p), axis=1, keepdims=True)
    e2 = jnp.min(jnp.where((p == p2) & (~sele1), ie, _NEXP), axis=1,
                 keepdims=True)
    denom = p1 + p2 + 1e-9
    w1 = p1 / denom
    w2 = p2 / denom

    acc = jnp.zeros_like(cont)
    for i in range(_NEXP):
        ex = _fdot(jax.nn.relu(_fdot(cont, eW1_ref[i]) + eb1_ref[i]),
                   eW2_ref[i]) + eb2_ref[i]
        wi = w1 * (e1 == i).astype(jnp.float32) + \
             w2 * (e2 == i).astype(jnp.float32)
        acc = acc + ex * wi

    rate = jax.nn.sigmoid(_fdot(acc, c2sW_ref[...]) + c2sb_ref[...])
    dec = _gif_stack(_fdot(rate * g, decW_ref[...]) + decb_ref[...])
    dec_ref[...] = dec


def _front(ids_col, emb_rows, enc_W, enc_b, s2c_W, s2c_b, rW1, rb1, rW2,
           rb2, eW1, eb1, eW2, eb2, c2s_W, c2s_b, dec_W, dec_b,
           interpret=False):
    return pl.pallas_call(
        _front_body,
        out_shape=jax.ShapeDtypeStruct((_S, _EMBED), jnp.float32),
        interpret=interpret,
    )(ids_col, emb_rows, enc_W, enc_b.reshape(1, -1), s2c_W,
      s2c_b.reshape(1, -1), rW1, rb1.reshape(1, -1), rW2,
      rb2.reshape(1, -1), eW1, eb1.reshape(_NEXP, 1, -1), eW2,
      eb2.reshape(_NEXP, 1, -1), c2s_W, c2s_b.reshape(1, -1), dec_W,
      dec_b.reshape(1, -1))


def _vocab_body(dec_ref, wt_ref, b_ref, o_ref):
    b_col = jnp.transpose(b_ref[...], (1, 0))
    o_ref[...] = lax.dot_general(
        wt_ref[...], dec_ref[...], (((1,), (1,)), ((), ())),
        preferred_element_type=jnp.float32) + b_col


def _vocab(dec, out_Wt, out_b2d, interpret=False):
    grid = (pl.cdiv(_VOCAB, _NBLK),)
    return pl.pallas_call(
        _vocab_body,
        grid=grid,
        in_specs=[
            pl.BlockSpec((_S, _EMBED), lambda j: (0, 0)),
            pl.BlockSpec((_NBLK, _EMBED), lambda j: (j, 0)),
            pl.BlockSpec((1, _NBLK), lambda j: (0, j)),
        ],
        out_specs=pl.BlockSpec((_NBLK, _S), lambda j: (j, 0)),
        out_shape=jax.ShapeDtypeStruct((_VOCAB, _S), jnp.float32),
        compiler_params=pltpu.CompilerParams(
            dimension_semantics=("arbitrary",),
            vmem_limit_bytes=56 * 1024 * 1024),
        interpret=interpret,
    )(dec, out_Wt, out_b2d)


def kernel(input_ids, emb, enc_W, enc_b, s2c_W, s2c_b, rW1, rb1, rW2, rb2,
           eW1, eb1, eW2, eb2, c2s_W, c2s_b, dec_W, dec_b, out_W, out_b):
    bsz, seq = input_ids.shape
    ids_flat = input_ids.reshape(-1)
    emb_rows = _sc_gather(_VOCAB, _EMBED, bsz * seq)(emb, ids_flat)
    dec = _front(ids_flat.reshape(-1, 1), emb_rows, enc_W, enc_b, s2c_W,
                 s2c_b, rW1, rb1, rW2, rb2, eW1, eb1, eW2, eb2, c2s_W,
                 c2s_b, dec_W, dec_b)
    logits_t = _vocab(dec, out_W.T, out_b.reshape(1, -1))
    return logits_t.T.reshape(bsz, seq, _VOCAB)


# speedup vs baseline: 3.1817x; 1.0082x over previous
import functools

import jax
import jax.numpy as jnp
from jax import lax
from jax.experimental import pallas as pl
from jax.experimental.pallas import tpu as pltpu
from jax.experimental.pallas import tpu_sc as plsc

_VOCAB = 100000
_EMBED = 128
_HIDDEN = 256
_MOE = 64
_NEXP = 8
_S = 2048
_L = 16

_NBLK = 2048


@functools.lru_cache(maxsize=None)
def _sc_gather(V, D, B):
    info = plsc.get_sparse_core_info()
    nw = info.num_cores * info.num_subcores
    bpw = B // nw
    mesh = plsc.VectorSubcoreMesh(core_axis_name="c", subcore_axis_name="s")

    @functools.partial(
        pl.kernel,
        mesh=mesh,
        out_type=jax.ShapeDtypeStruct((B, D), jnp.float32),
        scratch_types=[
            pltpu.VMEM((bpw,), jnp.int32),
            pltpu.VMEM((bpw, D), jnp.float32),
            pltpu.SemaphoreType.DMA,
        ],
        compiler_params=pltpu.CompilerParams(use_tc_tiling_on_sc=True),
    )
    def g(table_hbm, idx_hbm, out_hbm, idx_v, rows_v, sem):
        wid = lax.axis_index("s") * info.num_cores + lax.axis_index("c")
        base = wid * bpw
        pltpu.sync_copy(idx_hbm.at[pl.ds(base, bpw)], idx_v)
        pltpu.async_copy(table_hbm.at[idx_v], rows_v, sem).wait()
        pltpu.sync_copy(rows_v, out_hbm.at[pl.ds(base, bpw)])

    return g


def _fdot(a, b):
    return lax.dot_general(a, b, (((1,), (0,)), ((), ())),
                           preferred_element_type=jnp.float32)


def _gif_stack(h):
    h25 = 2.5 * h
    w = jnp.full_like(h, -2.5)
    th_sum = jnp.zeros_like(h)
    for _ in range(_L):
        w = w + h25
        th = jnp.tanh(w)
        th_sum = th_sum + th
        w = w - (1.25 * th + 1.25)
    return th_sum * (1.0 / (2 * _L)) + 0.5


def _front_compute(ids_ref, er_ref, encW_ref, encb_ref, s2cW_ref, s2cb_ref,
                   rW1_ref, rb1_ref, rW2_ref, rb2_ref, eW1_ref, eb1_ref,
                   eW2_ref, eb2_ref, c2sW_ref, c2sb_ref, decW_ref, decb_ref):
    ids = ids_ref[...]
    score = (ids % 97).astype(jnp.float32) * (1.0 / 97.0)
    it = lax.broadcasted_iota(jnp.int32, score.shape, 0)
    m1 = jnp.max(score, axis=0, keepdims=True)
    i1 = jnp.min(jnp.where(score == m1, it, _S), axis=0, keepdims=True)
    sel1 = it == i1
    m2 = jnp.max(jnp.where(sel1, -1.0, score), axis=0, keepdims=True)
    i2 = jnp.min(jnp.where((score == m2) & (~sel1), it, _S), axis=0,
                 keepdims=True)
    g = jnp.where(sel1 | (it == i2), 1.5, 1.0)

    mod = er_ref[...] * g
    spk = _gif_stack(_fdot(mod, encW_ref[...]) + encb_ref[...])
    cont = _fdot(spk, s2cW_ref[...]) + s2cb_ref[...]

    hr = jnp.tanh(_fdot(cont, rW1_ref[...]) + rb1_ref[...])
    rl = (_fdot(hr, rW2_ref[...]) + rb2_ref[...]) * g
    rmax = jnp.max(rl, axis=1, keepdims=True)
    e = jnp.exp(rl - rmax)
    p = e / jnp.sum(e, axis=1, keepdims=True)
    ie = lax.broadcasted_iota(jnp.int32, p.shape, 1)
    p1 = jnp.max(p, axis=1, keepdims=True)
    e1 = jnp.min(jnp.where(p == p1, ie, _NEXP), axis=1, keepdims=True)
    sele1 = ie == e1
    p2 = jnp.max(jnp.where(sele1, -1.0, p), axis=1, keepdims=True)
    e2 = jnp.min(jnp.where((p == p2) & (~sele1), ie, _NEXP), axis=1,
                 keepdims=True)
    denom = p1 + p2 + 1e-9
    w1 = p1 / denom
    w2 = p2 / denom

    acc = jnp.zeros_like(cont)
    for i in range(_NEXP):
        ex = _fdot(jax.nn.relu(_fdot(cont, eW1_ref[i]) + eb1_ref[i]),
                   eW2_ref[i]) + eb2_ref[i]
        wi = w1 * (e1 == i).astype(jnp.float32) + \
             w2 * (e2 == i).astype(jnp.float32)
        acc = acc + ex * wi

    rate = jax.nn.sigmoid(_fdot(acc, c2sW_ref[...]) + c2sb_ref[...])
    return _gif_stack(_fdot(rate * g, decW_ref[...]) + decb_ref[...])


def _fused_body(ids_ref, er_ref, encW_ref, encb_ref, s2cW_ref, s2cb_ref,
                rW1_ref, rb1_ref, rW2_ref, rb2_ref, eW1_ref, eb1_ref,
                eW2_ref, eb2_ref, c2sW_ref, c2sb_ref, decW_ref, decb_ref,
                wt_ref, b_ref, o_ref, dec_scr):
    @pl.when(pl.program_id(0) == 0)
    def _():
        dec_scr[...] = _front_compute(
            ids_ref, er_ref, encW_ref, encb_ref, s2cW_ref, s2cb_ref,
            rW1_ref, rb1_ref, rW2_ref, rb2_ref, eW1_ref, eb1_ref,
            eW2_ref, eb2_ref, c2sW_ref, c2sb_ref, decW_ref, decb_ref)

    b_col = jnp.transpose(b_ref[...], (1, 0))
    o_ref[...] = lax.dot_general(
        wt_ref[...], dec_scr[...], (((1,), (1,)), ((), ())),
        preferred_element_type=jnp.float32) + b_col


def _fused(ids_col, emb_rows, enc_W, enc_b, s2c_W, s2c_b, rW1, rb1, rW2,
           rb2, eW1, eb1, eW2, eb2, c2s_W, c2s_b, dec_W, dec_b, out_Wt,
           out_b2d, interpret=False):
    grid = (pl.cdiv(_VOCAB, _NBLK),)
    z2 = lambda j: (0, 0)
    z3 = lambda j: (0, 0, 0)
    front_specs = [
        pl.BlockSpec((_S, 1), z2),
        pl.BlockSpec((_S, _EMBED), z2),
        pl.BlockSpec((_EMBED, _HIDDEN), z2),
        pl.BlockSpec((1, _HIDDEN), z2),
        pl.BlockSpec((_HIDDEN, _MOE), z2),
        pl.BlockSpec((1, _MOE), z2),
        pl.BlockSpec((_MOE, 64), z2),
        pl.BlockSpec((1, 64), z2),
        pl.BlockSpec((64, _NEXP), z2),
        pl.BlockSpec((1, _NEXP), z2),
        pl.BlockSpec((_NEXP, _MOE, _HIDDEN // 2), z3),
        pl.BlockSpec((_NEXP, 1, _HIDDEN // 2), z3),
        pl.BlockSpec((_NEXP, _HIDDEN // 2, _MOE), z3),
        pl.BlockSpec((_NEXP, 1, _MOE), z3),
        pl.BlockSpec((_MOE, _HIDDEN), z2),
        pl.BlockSpec((1, _HIDDEN), z2),
        pl.BlockSpec((_HIDDEN, _EMBED), z2),
        pl.BlockSpec((1, _EMBED), z2),
    ]
    return pl.pallas_call(
        _fused_body,
        grid=grid,
        in_specs=front_specs + [
            pl.BlockSpec((_NBLK, _EMBED), lambda j: (j, 0)),
            pl.BlockSpec((1, _NBLK), lambda j: (0, j)),
        ],
        out_specs=pl.BlockSpec((_NBLK, _S), lambda j: (j, 0)),
        out_shape=jax.ShapeDtypeStruct((_VOCAB, _S), jnp.float32),
        scratch_shapes=[pltpu.VMEM((_S, _EMBED), jnp.float32)],
        compiler_params=pltpu.CompilerParams(
            dimension_semantics=("arbitrary",),
            vmem_limit_bytes=56 * 1024 * 1024),
        interpret=interpret,
    )(ids_col, emb_rows, enc_W, enc_b.reshape(1, -1), s2c_W,
      s2c_b.reshape(1, -1), rW1, rb1.reshape(1, -1), rW2,
      rb2.reshape(1, -1), eW1, eb1.reshape(_NEXP, 1, -1), eW2,
      eb2.reshape(_NEXP, 1, -1), c2s_W, c2s_b.reshape(1, -1), dec_W,
      dec_b.reshape(1, -1), out_Wt, out_b2d)


def kernel(input_ids, emb, enc_W, enc_b, s2c_W, s2c_b, rW1, rb1, rW2, rb2,
           eW1, eb1, eW2, eb2, c2s_W, c2s_b, dec_W, dec_b, out_W, out_b):
    bsz, seq = input_ids.shape
    ids_flat = input_ids.reshape(-1)
    emb_rows = _sc_gather(_VOCAB, _EMBED, bsz * seq)(emb, ids_flat)
    logits_t = _fused(ids_flat.reshape(-1, 1), emb_rows, enc_W, enc_b,
                      s2c_W, s2c_b, rW1, rb1, rW2, rb2, eW1, eb1, eW2,
                      eb2, c2s_W, c2s_b, dec_W, dec_b, out_W.T,
                      out_b.reshape(1, -1))
    return logits_t.T.reshape(bsz, seq, _VOCAB)
